# Initial kernel scaffold; baseline (speedup 1.0000x reference)
import jax
import jax.numpy as jnp
from jax.experimental import pallas as pl


def kernel(x, edge_index, batch, W1, b1, W2, b2):
    raise NotImplementedError("write your pallas kernel here")


# speedup vs baseline: 14.4005x; 14.4005x over previous
import functools

import jax
import jax.numpy as jnp
from jax import lax
from jax.experimental import pallas as pl
from jax.experimental.pallas import tpu as pltpu
from jax.experimental.pallas import tpu_sc as plsc

N = 10000
E = 320000
D = 128
G = 64

NC = 2
NS = 16
NW = NC * NS
EPW = E // NW
K = 80
NCHUNK = EPW // K
RPS = N // NS
ZR = 80

BLK = 1000
GRID = N // BLK


def _sc_mesh():
    return plsc.VectorSubcoreMesh(
        core_axis_name="c", subcore_axis_name="s", num_cores=NC, num_subcores=NS
    )


def _deg_partials_body(dst_hbm, deg_out, idx_v, deg_v):
    c = lax.axis_index("c")
    s = lax.axis_index("s")
    wid = s * NC + c

    def zero_body(t, carry):
        deg_v[0, pl.ds(t * 16, 16)] = jnp.zeros((16,), jnp.float32)
        return carry

    lax.fori_loop(0, N // 16, zero_body, 0)

    ones16 = jnp.ones((16,), jnp.float32)
    zeros16 = jnp.zeros((16,), jnp.int32)
    base = wid * EPW

    def stage_body(q, carry):
        pltpu.sync_copy(dst_hbm.at[pl.ds(base + q * 2000, 2000)], idx_v)

        def hist_body(t, c2):
            idx16 = idx_v[pl.ds(t * 16, 16)]
            plsc.addupdate_scatter(deg_v, [zeros16, idx16], ones16)
            return c2

        lax.fori_loop(0, 125, hist_body, 0)
        return carry

    lax.fori_loop(0, EPW // 2000, stage_body, 0)
    pltpu.sync_copy(deg_v, deg_out.at[wid])


def _deg_partials(dst):
    return pl.kernel(
        _deg_partials_body,
        out_type=jax.ShapeDtypeStruct((NW, 1, N), jnp.float32),
        mesh=_sc_mesh(),
        compiler_params=pltpu.CompilerParams(needs_layout_passes=False),
        scratch_types=[
            pltpu.VMEM((2000,), jnp.int32),
            pltpu.VMEM((1, N), jnp.float32),
        ],
    )(dst)


def _edge_pass_body(y_hbm, src_hbm, dst_hbm, out_hbm, sidx, didx, rows, zrow, acc, sem):
    c = lax.axis_index("c")
    s = lax.axis_index("s")
    wid = s * NC + c

    def zbuf_body(i, carry):
        for j in range(D // 16):
            zrow[i, pl.ds(j * 16, 16)] = jnp.zeros((16,), jnp.float32)
        return carry

    lax.fori_loop(0, ZR, zbuf_body, 0)

    @pl.when(s < NS - 1)
    def _():
        def zc(q, carry):
            pltpu.sync_copy(zrow, acc.at[pl.ds(s * 640 + q * ZR, ZR)])
            return carry

        lax.fori_loop(0, 640 // ZR, zc, 0)

    @pl.when(s == NS - 1)
    def _():
        def zc(q, carry):
            pltpu.sync_copy(zrow, acc.at[pl.ds((NS - 1) * 640 + q * ZR, ZR)])
            return carry

        lax.fori_loop(0, (N - (NS - 1) * 640) // ZR, zc, 0)

    plsc.subcore_barrier()

    base = wid * EPW

    def chunk_body(t, carry):
        pltpu.sync_copy(src_hbm.at[pl.ds(base + t * K, K)], sidx)
        pltpu.sync_copy(dst_hbm.at[pl.ds(base + t * K, K)], didx)
        pltpu.async_copy(y_hbm.at[sidx], rows, sem).wait()
        pltpu.sync_copy(rows, acc.at[didx], add=True)
        return carry

    lax.fori_loop(0, NCHUNK, chunk_body, 0)
    plsc.subcore_barrier()

    @pl.when(s < NS - 1)
    def _():
        pltpu.sync_copy(
            acc.at[pl.ds(s * 640, 640)], out_hbm.at[c, pl.ds(s * 640, 640)]
        )

    @pl.when(s == NS - 1)
    def _():
        pltpu.sync_copy(
            acc.at[pl.ds((NS - 1) * 640, N - (NS - 1) * 640)],
            out_hbm.at[c, pl.ds((NS - 1) * 640, N - (NS - 1) * 640)],
        )


def _edge_pass(y, src, dst):
    return pl.kernel(
        _edge_pass_body,
        out_type=jax.ShapeDtypeStruct((NC, N, D), jnp.float32),
        mesh=_sc_mesh(),
        compiler_params=pltpu.CompilerParams(needs_layout_passes=False),
        scratch_types=[
            pltpu.VMEM((K,), jnp.int32),
            pltpu.VMEM((K,), jnp.int32),
            pltpu.VMEM((K, D), jnp.float32),
            pltpu.VMEM((ZR, D), jnp.float32),
            pltpu.VMEM_SHARED((N, D), jnp.float32),
            pltpu.SemaphoreType.DMA,
        ],
    )(y, src, dst)


def _tc_norm_mm_body(x_ref, w_ref, degp_ref, y_ref, dinv_ref):
    deg = 1.0 + jnp.sum(degp_ref[...], axis=1, keepdims=True)
    dinv = lax.rsqrt(jnp.maximum(deg, 1e-12))
    xw = jnp.dot(x_ref[...], w_ref[...], preferred_element_type=jnp.float32)
    y_ref[...] = xw * dinv
    dinv_ref[...] = dinv


def _tc_norm_mm(x, w1, degp_t):
    return pl.pallas_call(
        _tc_norm_mm_body,
        grid=(GRID,),
        in_specs=[
            pl.BlockSpec((BLK, D), lambda i: (i, 0)),
            pl.BlockSpec((D, D), lambda i: (0, 0)),
            pl.BlockSpec((BLK, NW), lambda i: (i, 0)),
        ],
        out_specs=[
            pl.BlockSpec((BLK, D), lambda i: (i, 0)),
            pl.BlockSpec((BLK, 1), lambda i: (i, 0)),
        ],
        out_shape=[
            jax.ShapeDtypeStruct((N, D), jnp.float32),
            jax.ShapeDtypeStruct((N, 1), jnp.float32),
        ],
    )(x, w1, degp_t)


def _tc_mid_body(acc_ref, y1_ref, dinv_ref, b1_ref, w2_ref, y2_ref):
    a = acc_ref[0] + acc_ref[1] + y1_ref[...]
    dinv = dinv_ref[...]
    h = jnp.maximum(a * dinv + b1_ref[...], 0.0)
    y2_ref[...] = jnp.dot(h, w2_ref[...], preferred_element_type=jnp.float32) * dinv


def _tc_mid(acc, y1, dinv, b1_2d, w2):
    return pl.pallas_call(
        _tc_mid_body,
        grid=(GRID,),
        in_specs=[
            pl.BlockSpec((NC, BLK, D), lambda i: (0, i, 0)),
            pl.BlockSpec((BLK, D), lambda i: (i, 0)),
            pl.BlockSpec((BLK, 1), lambda i: (i, 0)),
            pl.BlockSpec((1, D), lambda i: (0, 0)),
            pl.BlockSpec((D, D), lambda i: (0, 0)),
        ],
        out_specs=pl.BlockSpec((BLK, D), lambda i: (i, 0)),
        out_shape=jax.ShapeDtypeStruct((N, D), jnp.float32),
    )(acc, y1, dinv, b1_2d, w2)


def _tc_pool_body(acc_ref, y2_ref, dinv_ref, b2_ref, batch_ref, out_ref, cnt_ref):
    i = pl.program_id(0)
    a = acc_ref[0] + acc_ref[1] + y2_ref[...]
    h = a * dinv_ref[...] + b2_ref[...]
    bt = batch_ref[0]
    oh_t = jnp.where(
        lax.broadcasted_iota(jnp.int32, (G, BLK), 0) == bt, 1.0, 0.0
    )
    sums = jnp.dot(oh_t, h, preferred_element_type=jnp.float32)
    cnts = jnp.dot(oh_t, jnp.ones((BLK, D), jnp.float32),
                   preferred_element_type=jnp.float32)

    @pl.when(i == 0)
    def _():
        out_ref[...] = jnp.zeros((G, D), jnp.float32)
        cnt_ref[...] = jnp.zeros((G, D), jnp.float32)

    out_ref[...] += sums
    cnt_ref[...] += cnts

    @pl.when(i == GRID - 1)
    def _():
        out_ref[...] = out_ref[...] / jnp.maximum(cnt_ref[...], 1.0)


def _tc_pool(acc, y2, dinv, b2_2d, batch_2d):
    return pl.pallas_call(
        _tc_pool_body,
        grid=(GRID,),
        in_specs=[
            pl.BlockSpec((NC, BLK, D), lambda i: (0, i, 0)),
            pl.BlockSpec((BLK, D), lambda i: (i, 0)),
            pl.BlockSpec((BLK, 1), lambda i: (i, 0)),
            pl.BlockSpec((1, D), lambda i: (0, 0)),
            pl.BlockSpec((1, 1, BLK), lambda i: (i, 0, 0)),
        ],
        out_specs=pl.BlockSpec((G, D), lambda i: (0, 0)),
        out_shape=jax.ShapeDtypeStruct((G, D), jnp.float32),
        scratch_shapes=[pltpu.VMEM((G, D), jnp.float32)],
    )(acc, y2, dinv, b2_2d, batch_2d)


def kernel(x, edge_index, batch, W1, b1, W2, b2):
    src = edge_index[0]
    dst = edge_index[1]

    degp = _deg_partials(dst)
    degp_t = degp.reshape(NW, N).T

    y1, dinv = _tc_norm_mm(x, W1, degp_t)
    acc1 = _edge_pass(y1, src, dst)
    y2 = _tc_mid(acc1, y1, dinv, b1.reshape(1, D), W2)
    acc2 = _edge_pass(y2, src, dst)
    return _tc_pool(acc2, y2, dinv, b2.reshape(1, D), batch.reshape(GRID, 1, BLK))


# speedup vs baseline: 27.1927x; 1.8883x over previous
import functools

import jax
import jax.numpy as jnp
from jax import lax
from jax.experimental import pallas as pl
from jax.experimental.pallas import tpu as pltpu
from jax.experimental.pallas import tpu_sc as plsc

N = 10000
E = 320000
D = 128
G = 64

NC = 2
NS = 16
NW = NC * NS
EPW = E // NW
K = 80
NCHUNK = EPW // K
RPS = N // NS
ZR = 80

BLK = 1000
GRID = N // BLK


def _sc_mesh():
    return plsc.VectorSubcoreMesh(
        core_axis_name="c", subcore_axis_name="s", num_cores=NC, num_subcores=NS
    )


def _deg_partials_body(dst_hbm, deg_out, idx_v, deg_v):
    c = lax.axis_index("c")
    s = lax.axis_index("s")
    wid = s * NC + c

    def zero_body(t, carry):
        deg_v[0, pl.ds(t * 16, 16)] = jnp.zeros((16,), jnp.float32)
        return carry

    lax.fori_loop(0, N // 16, zero_body, 0)

    ones16 = jnp.ones((16,), jnp.float32)
    zeros16 = jnp.zeros((16,), jnp.int32)
    base = wid * EPW

    def stage_body(q, carry):
        pltpu.sync_copy(dst_hbm.at[pl.ds(base + q * 2000, 2000)], idx_v)

        def hist_body(t, c2):
            idx16 = idx_v[pl.ds(t * 16, 16)]
            plsc.addupdate_scatter(deg_v, [zeros16, idx16], ones16)
            return c2

        lax.fori_loop(0, 125, hist_body, 0)
        return carry

    lax.fori_loop(0, EPW // 2000, stage_body, 0)
    pltpu.sync_copy(deg_v, deg_out.at[wid])


def _deg_partials(dst):
    return pl.kernel(
        _deg_partials_body,
        out_type=jax.ShapeDtypeStruct((NW, 1, N), jnp.float32),
        mesh=_sc_mesh(),
        compiler_params=pltpu.CompilerParams(needs_layout_passes=False),
        scratch_types=[
            pltpu.VMEM((2000,), jnp.int32),
            pltpu.VMEM((1, N), jnp.float32),
        ],
    )(dst)


def _edge_pass_body(
    y_hbm, src_hbm, dst_hbm, out_hbm,
    sidx0, didx0, rows0, semi0, semg0,
    sidx1, didx1, rows1, semi1, semg1,
    zrow, acc,
):
    c = lax.axis_index("c")
    s = lax.axis_index("s")
    wid = s * NC + c

    def zbuf_body(i, carry):
        for j in range(D // 16):
            zrow[i, pl.ds(j * 16, 16)] = jnp.zeros((16,), jnp.float32)
        return carry

    lax.fori_loop(0, ZR, zbuf_body, 0)

    @pl.when(s < NS - 1)
    def _():
        def zc(q, carry):
            pltpu.sync_copy(zrow, acc.at[pl.ds(s * 640 + q * ZR, ZR)])
            return carry

        lax.fori_loop(0, 640 // ZR, zc, 0)

    @pl.when(s == NS - 1)
    def _():
        def zc(q, carry):
            pltpu.sync_copy(zrow, acc.at[pl.ds((NS - 1) * 640 + q * ZR, ZR)])
            return carry

        lax.fori_loop(0, (N - (NS - 1) * 640) // ZR, zc, 0)

    plsc.subcore_barrier()

    base = wid * EPW
    bufs = (
        (sidx0, didx0, rows0, semi0, semg0),
        (sidx1, didx1, rows1, semi1, semg1),
    )

    def idx_start(t, buf):
        si, di, _, smi, _ = buf
        pltpu.async_copy(src_hbm.at[pl.ds(base + t * K, K)], si, smi)
        pltpu.async_copy(dst_hbm.at[pl.ds(base + t * K, K)], di, smi)

    def idx_wait(t, buf):
        si, di, _, smi, _ = buf
        pltpu.make_async_copy(src_hbm.at[pl.ds(base + t * K, K)], si, smi).wait()
        pltpu.make_async_copy(dst_hbm.at[pl.ds(base + t * K, K)], di, smi).wait()

    def gather_start(buf):
        si, _, ri, _, smg = buf
        pltpu.async_copy(y_hbm.at[si], ri, smg)

    def gather_wait(buf):
        si, _, ri, _, smg = buf
        pltpu.make_async_copy(y_hbm.at[si], ri, smg).wait()

    def scatter(buf):
        _, di, ri, _, _ = buf
        pltpu.sync_copy(ri, acc.at[di], add=True)

    idx_start(0, bufs[0])
    idx_start(1, bufs[1])
    idx_wait(0, bufs[0])
    gather_start(bufs[0])

    def chunk_body(g, carry):
        for b in range(2):
            t = 2 * g + b
            cur = bufs[b]
            nxt = bufs[1 - b]
            idx_wait(t + 1, nxt)
            gather_start(nxt)
            gather_wait(cur)
            scatter(cur)

            @pl.when(t + 2 < NCHUNK)
            def _():
                idx_start(t + 2, cur)

        return carry

    lax.fori_loop(0, (NCHUNK - 1) // 2, chunk_body, 0)
    gather_wait(bufs[(NCHUNK - 1) % 2])
    scatter(bufs[(NCHUNK - 1) % 2])
    plsc.subcore_barrier()

    @pl.when(s < NS - 1)
    def _():
        pltpu.sync_copy(
            acc.at[pl.ds(s * 640, 640)], out_hbm.at[c, pl.ds(s * 640, 640)]
        )

    @pl.when(s == NS - 1)
    def _():
        pltpu.sync_copy(
            acc.at[pl.ds((NS - 1) * 640, N - (NS - 1) * 640)],
            out_hbm.at[c, pl.ds((NS - 1) * 640, N - (NS - 1) * 640)],
        )


def _edge_pass(y, src, dst):
    return pl.kernel(
        _edge_pass_body,
        out_type=jax.ShapeDtypeStruct((NC, N, D), jnp.float32),
        mesh=_sc_mesh(),
        compiler_params=pltpu.CompilerParams(needs_layout_passes=False),
        scratch_types=[
            pltpu.VMEM((K,), jnp.int32),
            pltpu.VMEM((K,), jnp.int32),
            pltpu.VMEM((K, D), jnp.float32),
            pltpu.SemaphoreType.DMA,
            pltpu.SemaphoreType.DMA,
            pltpu.VMEM((K,), jnp.int32),
            pltpu.VMEM((K,), jnp.int32),
            pltpu.VMEM((K, D), jnp.float32),
            pltpu.SemaphoreType.DMA,
            pltpu.SemaphoreType.DMA,
            pltpu.VMEM((ZR, D), jnp.float32),
            pltpu.VMEM_SHARED((N, D), jnp.float32),
        ],
    )(y, src, dst)


def _tc_norm_mm_body(x_ref, w_ref, degp_ref, y_ref, dinv_ref):
    deg = 1.0 + jnp.sum(degp_ref[...], axis=1, keepdims=True)
    dinv = lax.rsqrt(jnp.maximum(deg, 1e-12))
    xw = jnp.dot(x_ref[...], w_ref[...], preferred_element_type=jnp.float32)
    y_ref[...] = xw * dinv
    dinv_ref[...] = dinv


def _tc_norm_mm(x, w1, degp_t):
    return pl.pallas_call(
        _tc_norm_mm_body,
        grid=(GRID,),
        in_specs=[
            pl.BlockSpec((BLK, D), lambda i: (i, 0)),
            pl.BlockSpec((D, D), lambda i: (0, 0)),
            pl.BlockSpec((BLK, NW), lambda i: (i, 0)),
        ],
        out_specs=[
            pl.BlockSpec((BLK, D), lambda i: (i, 0)),
            pl.BlockSpec((BLK, 1), lambda i: (i, 0)),
        ],
        out_shape=[
            jax.ShapeDtypeStruct((N, D), jnp.float32),
            jax.ShapeDtypeStruct((N, 1), jnp.float32),
        ],
    )(x, w1, degp_t)


def _tc_mid_body(acc_ref, y1_ref, dinv_ref, b1_ref, w2_ref, y2_ref):
    a = acc_ref[0] + acc_ref[1] + y1_ref[...]
    dinv = dinv_ref[...]
    h = jnp.maximum(a * dinv + b1_ref[...], 0.0)
    y2_ref[...] = jnp.dot(h, w2_ref[...], preferred_element_type=jnp.float32) * dinv


def _tc_mid(acc, y1, dinv, b1_2d, w2):
    return pl.pallas_call(
        _tc_mid_body,
        grid=(GRID,),
        in_specs=[
            pl.BlockSpec((NC, BLK, D), lambda i: (0, i, 0)),
            pl.BlockSpec((BLK, D), lambda i: (i, 0)),
            pl.BlockSpec((BLK, 1), lambda i: (i, 0)),
            pl.BlockSpec((1, D), lambda i: (0, 0)),
            pl.BlockSpec((D, D), lambda i: (0, 0)),
        ],
        out_specs=pl.BlockSpec((BLK, D), lambda i: (i, 0)),
        out_shape=jax.ShapeDtypeStruct((N, D), jnp.float32),
    )(acc, y1, dinv, b1_2d, w2)


def _tc_pool_body(acc_ref, y2_ref, dinv_ref, b2_ref, batch_ref, out_ref, cnt_ref):
    i = pl.program_id(0)
    a = acc_ref[0] + acc_ref[1] + y2_ref[...]
    h = a * dinv_ref[...] + b2_ref[...]
    bt = batch_ref[0]
    oh_t = jnp.where(
        lax.broadcasted_iota(jnp.int32, (G, BLK), 0) == bt, 1.0, 0.0
    )
    sums = jnp.dot(oh_t, h, preferred_element_type=jnp.float32)
    cnts = jnp.dot(oh_t, jnp.ones((BLK, D), jnp.float32),
                   preferred_element_type=jnp.float32)

    @pl.when(i == 0)
    def _():
        out_ref[...] = jnp.zeros((G, D), jnp.float32)
        cnt_ref[...] = jnp.zeros((G, D), jnp.float32)

    out_ref[...] += sums
    cnt_ref[...] += cnts

    @pl.when(i == GRID - 1)
    def _():
        out_ref[...] = out_ref[...] / jnp.maximum(cnt_ref[...], 1.0)


def _tc_pool(acc, y2, dinv, b2_2d, batch_2d):
    return pl.pallas_call(
        _tc_pool_body,
        grid=(GRID,),
        in_specs=[
            pl.BlockSpec((NC, BLK, D), lambda i: (0, i, 0)),
            pl.BlockSpec((BLK, D), lambda i: (i, 0)),
            pl.BlockSpec((BLK, 1), lambda i: (i, 0)),
            pl.BlockSpec((1, D), lambda i: (0, 0)),
            pl.BlockSpec((1, 1, BLK), lambda i: (i, 0, 0)),
        ],
        out_specs=pl.BlockSpec((G, D), lambda i: (0, 0)),
        out_shape=jax.ShapeDtypeStruct((G, D), jnp.float32),
        scratch_shapes=[pltpu.VMEM((G, D), jnp.float32)],
    )(acc, y2, dinv, b2_2d, batch_2d)


def kernel(x, edge_index, batch, W1, b1, W2, b2):
    src = edge_index[0]
    dst = edge_index[1]

    degp = _deg_partials(dst)
    degp_t = degp.reshape(NW, N).T

    y1, dinv = _tc_norm_mm(x, W1, degp_t)
    acc1 = _edge_pass(y1, src, dst)
    y2 = _tc_mid(acc1, y1, dinv, b1.reshape(1, D), W2)
    acc2 = _edge_pass(y2, src, dst)
    return _tc_pool(acc2, y2, dinv, b2.reshape(1, D), batch.reshape(GRID, 1, BLK))


# speedup vs baseline: 28.7751x; 1.0582x over previous
import functools

import jax
import jax.numpy as jnp
from jax import lax
from jax.experimental import pallas as pl
from jax.experimental.pallas import tpu as pltpu
from jax.experimental.pallas import tpu_sc as plsc

N = 10000
E = 320000
D = 128
G = 64

NC = 2
NS = 16
NW = NC * NS
EPW = E // NW
K = 80
NCHUNK = EPW // K
RPS = N // NS
ZR = 80

BLK = 1000
GRID = N // BLK


def _sc_mesh():
    return plsc.VectorSubcoreMesh(
        core_axis_name="c", subcore_axis_name="s", num_cores=NC, num_subcores=NS
    )


def _deg_partials_body(dst_hbm, deg_out, idx_v, deg_v):
    c = lax.axis_index("c")
    s = lax.axis_index("s")
    wid = s * NC + c

    def zero_body(t, carry):
        deg_v[0, pl.ds(t * 16, 16)] = jnp.zeros((16,), jnp.float32)
        return carry

    lax.fori_loop(0, N // 16, zero_body, 0)

    ones16 = jnp.ones((16,), jnp.float32)
    zeros16 = jnp.zeros((16,), jnp.int32)
    base = wid * EPW

    def stage_body(q, carry):
        pltpu.sync_copy(dst_hbm.at[pl.ds(base + q * 2000, 2000)], idx_v)

        def hist_body(t, c2):
            idx16 = idx_v[pl.ds(t * 16, 16)]
            plsc.addupdate_scatter(deg_v, [zeros16, idx16], ones16)
            return c2

        lax.fori_loop(0, 125, hist_body, 0)
        return carry

    lax.fori_loop(0, EPW // 2000, stage_body, 0)
    pltpu.sync_copy(deg_v, deg_out.at[wid])


def _deg_partials(dst):
    return pl.kernel(
        _deg_partials_body,
        out_type=jax.ShapeDtypeStruct((NW, 1, N), jnp.float32),
        mesh=_sc_mesh(),
        compiler_params=pltpu.CompilerParams(needs_layout_passes=False),
        scratch_types=[
            pltpu.VMEM((2000,), jnp.int32),
            pltpu.VMEM((1, N), jnp.float32),
        ],
    )(dst)


def _edge_pass_body(
    y_hbm, src_hbm, dst_hbm, out_hbm,
    sidx0, didx0, rows0, semi0, semg0,
    sidx1, didx1, rows1, semi1, semg1,
    sidx2, didx2, rows2, semi2, semg2,
    zrow, acc,
):
    c = lax.axis_index("c")
    s = lax.axis_index("s")
    wid = s * NC + c

    def zbuf_body(i, carry):
        for j in range(D // 16):
            zrow[i, pl.ds(j * 16, 16)] = jnp.zeros((16,), jnp.float32)
        return carry

    lax.fori_loop(0, ZR, zbuf_body, 0)

    @pl.when(s < NS - 1)
    def _():
        def zc(q, carry):
            pltpu.sync_copy(zrow, acc.at[pl.ds(s * 640 + q * ZR, ZR)])
            return carry

        lax.fori_loop(0, 640 // ZR, zc, 0)

    @pl.when(s == NS - 1)
    def _():
        def zc(q, carry):
            pltpu.sync_copy(zrow, acc.at[pl.ds((NS - 1) * 640 + q * ZR, ZR)])
            return carry

        lax.fori_loop(0, (N - (NS - 1) * 640) // ZR, zc, 0)

    plsc.subcore_barrier()

    base = wid * EPW
    bufs = (
        (sidx0, didx0, rows0, semi0, semg0),
        (sidx1, didx1, rows1, semi1, semg1),
        (sidx2, didx2, rows2, semi2, semg2),
    )

    def idx_start(t, buf):
        si, di, _, smi, _ = buf
        pltpu.async_copy(src_hbm.at[pl.ds(base + t * K, K)], si, smi)
        pltpu.async_copy(dst_hbm.at[pl.ds(base + t * K, K)], di, smi)

    def idx_wait(t, buf):
        si, di, _, smi, _ = buf
        pltpu.make_async_copy(src_hbm.at[pl.ds(base + t * K, K)], si, smi).wait()
        pltpu.make_async_copy(dst_hbm.at[pl.ds(base + t * K, K)], di, smi).wait()

    def gather_start(buf):
        si, _, ri, _, smg = buf
        pltpu.async_copy(y_hbm.at[si], ri, smg)

    def gather_wait(buf):
        si, _, ri, _, smg = buf
        pltpu.make_async_copy(y_hbm.at[si], ri, smg).wait()

    def scatter(buf):
        _, di, ri, _, _ = buf
        pltpu.sync_copy(ri, acc.at[di], add=True)

    for p in range(3):
        idx_start(p, bufs[p])
    idx_wait(0, bufs[0])
    gather_start(bufs[0])
    idx_wait(1, bufs[1])
    gather_start(bufs[1])

    def chunk_body(g, carry):
        for b in range(3):
            t = 3 * g + b
            cur = bufs[b]
            nx2 = bufs[(b + 2) % 3]
            idx_wait(t + 2, nx2)
            gather_start(nx2)
            gather_wait(cur)
            scatter(cur)

            @pl.when(t + 3 < NCHUNK)
            def _():
                idx_start(t + 3, cur)

        return carry

    lax.fori_loop(0, (NCHUNK - 2) // 3, chunk_body, 0)
    for t in (NCHUNK - 2, NCHUNK - 1):
        gather_wait(bufs[t % 3])
        scatter(bufs[t % 3])
    plsc.subcore_barrier()

    @pl.when(s < NS - 1)
    def _():
        pltpu.sync_copy(
            acc.at[pl.ds(s * 640, 640)], out_hbm.at[c, pl.ds(s * 640, 640)]
        )

    @pl.when(s == NS - 1)
    def _():
        pltpu.sync_copy(
            acc.at[pl.ds((NS - 1) * 640, N - (NS - 1) * 640)],
            out_hbm.at[c, pl.ds((NS - 1) * 640, N - (NS - 1) * 640)],
        )


def _edge_pass(y, src, dst):
    return pl.kernel(
        _edge_pass_body,
        out_type=jax.ShapeDtypeStruct((NC, N, D), jnp.float32),
        mesh=_sc_mesh(),
        compiler_params=pltpu.CompilerParams(needs_layout_passes=False),
        scratch_types=[
            pltpu.VMEM((K,), jnp.int32),
            pltpu.VMEM((K,), jnp.int32),
            pltpu.VMEM((K, D), jnp.float32),
            pltpu.SemaphoreType.DMA,
            pltpu.SemaphoreType.DMA,
            pltpu.VMEM((K,), jnp.int32),
            pltpu.VMEM((K,), jnp.int32),
            pltpu.VMEM((K, D), jnp.float32),
            pltpu.SemaphoreType.DMA,
            pltpu.SemaphoreType.DMA,
            pltpu.VMEM((K,), jnp.int32),
            pltpu.VMEM((K,), jnp.int32),
            pltpu.VMEM((K, D), jnp.float32),
            pltpu.SemaphoreType.DMA,
            pltpu.SemaphoreType.DMA,
            pltpu.VMEM((ZR, D), jnp.float32),
            pltpu.VMEM_SHARED((N, D), jnp.float32),
        ],
    )(y, src, dst)


def _tc_norm_mm_body(x_ref, w_ref, degp_ref, y_ref, dinv_ref):
    deg = 1.0 + jnp.sum(degp_ref[...], axis=1, keepdims=True)
    dinv = lax.rsqrt(jnp.maximum(deg, 1e-12))
    xw = jnp.dot(x_ref[...], w_ref[...], preferred_element_type=jnp.float32)
    y_ref[...] = xw * dinv
    dinv_ref[...] = dinv


def _tc_norm_mm(x, w1, degp_t):
    return pl.pallas_call(
        _tc_norm_mm_body,
        grid=(GRID,),
        in_specs=[
            pl.BlockSpec((BLK, D), lambda i: (i, 0)),
            pl.BlockSpec((D, D), lambda i: (0, 0)),
            pl.BlockSpec((BLK, NW), lambda i: (i, 0)),
        ],
        out_specs=[
            pl.BlockSpec((BLK, D), lambda i: (i, 0)),
            pl.BlockSpec((BLK, 1), lambda i: (i, 0)),
        ],
        out_shape=[
            jax.ShapeDtypeStruct((N, D), jnp.float32),
            jax.ShapeDtypeStruct((N, 1), jnp.float32),
        ],
    )(x, w1, degp_t)


def _tc_mid_body(acc_ref, y1_ref, dinv_ref, b1_ref, w2_ref, y2_ref):
    a = acc_ref[0] + acc_ref[1] + y1_ref[...]
    dinv = dinv_ref[...]
    h = jnp.maximum(a * dinv + b1_ref[...], 0.0)
    y2_ref[...] = jnp.dot(h, w2_ref[...], preferred_element_type=jnp.float32) * dinv


def _tc_mid(acc, y1, dinv, b1_2d, w2):
    return pl.pallas_call(
        _tc_mid_body,
        grid=(GRID,),
        in_specs=[
            pl.BlockSpec((NC, BLK, D), lambda i: (0, i, 0)),
            pl.BlockSpec((BLK, D), lambda i: (i, 0)),
            pl.BlockSpec((BLK, 1), lambda i: (i, 0)),
            pl.BlockSpec((1, D), lambda i: (0, 0)),
            pl.BlockSpec((D, D), lambda i: (0, 0)),
        ],
        out_specs=pl.BlockSpec((BLK, D), lambda i: (i, 0)),
        out_shape=jax.ShapeDtypeStruct((N, D), jnp.float32),
    )(acc, y1, dinv, b1_2d, w2)


def _tc_pool_body(acc_ref, y2_ref, dinv_ref, b2_ref, batch_ref, out_ref, cnt_ref):
    i = pl.program_id(0)
    a = acc_ref[0] + acc_ref[1] + y2_ref[...]
    h = a * dinv_ref[...] + b2_ref[...]
    bt = batch_ref[0]
    oh_t = jnp.where(
        lax.broadcasted_iota(jnp.int32, (G, BLK), 0) == bt, 1.0, 0.0
    )
    sums = jnp.dot(oh_t, h, preferred_element_type=jnp.float32)
    cnts = jnp.dot(oh_t, jnp.ones((BLK, D), jnp.float32),
                   preferred_element_type=jnp.float32)

    @pl.when(i == 0)
    def _():
        out_ref[...] = jnp.zeros((G, D), jnp.float32)
        cnt_ref[...] = jnp.zeros((G, D), jnp.float32)

    out_ref[...] += sums
    cnt_ref[...] += cnts

    @pl.when(i == GRID - 1)
    def _():
        out_ref[...] = out_ref[...] / jnp.maximum(cnt_ref[...], 1.0)


def _tc_pool(acc, y2, dinv, b2_2d, batch_2d):
    return pl.pallas_call(
        _tc_pool_body,
        grid=(GRID,),
        in_specs=[
            pl.BlockSpec((NC, BLK, D), lambda i: (0, i, 0)),
            pl.BlockSpec((BLK, D), lambda i: (i, 0)),
            pl.BlockSpec((BLK, 1), lambda i: (i, 0)),
            pl.BlockSpec((1, D), lambda i: (0, 0)),
            pl.BlockSpec((1, 1, BLK), lambda i: (i, 0, 0)),
        ],
        out_specs=pl.BlockSpec((G, D), lambda i: (0, 0)),
        out_shape=jax.ShapeDtypeStruct((G, D), jnp.float32),
        scratch_shapes=[pltpu.VMEM((G, D), jnp.float32)],
    )(acc, y2, dinv, b2_2d, batch_2d)


def kernel(x, edge_index, batch, W1, b1, W2, b2):
    src = edge_index[0]
    dst = edge_index[1]

    degp = _deg_partials(dst)
    degp_t = degp.reshape(NW, N).T

    y1, dinv = _tc_norm_mm(x, W1, degp_t)
    acc1 = _edge_pass(y1, src, dst)
    y2 = _tc_mid(acc1, y1, dinv, b1.reshape(1, D), W2)
    acc2 = _edge_pass(y2, src, dst)
    return _tc_pool(acc2, y2, dinv, b2.reshape(1, D), batch.reshape(GRID, 1, BLK))


# speedup vs baseline: 31.1710x; 1.0833x over previous
import functools

import jax
import jax.numpy as jnp
from jax import lax
from jax.experimental import pallas as pl
from jax.experimental.pallas import tpu as pltpu
from jax.experimental.pallas import tpu_sc as plsc

N = 10000
E = 320000
D = 128
G = 64

NC = 2
NS = 16
NW = NC * NS
EPW = E // NW
K = 128
NFULL = EPW // K
KM = EPW - NFULL * K
ZR = 40

BLK = 1000
GRID = N // BLK


def _sc_mesh():
    return plsc.VectorSubcoreMesh(
        core_axis_name="c", subcore_axis_name="s", num_cores=NC, num_subcores=NS
    )


def _deg_partials_body(dst_hbm, deg_out, idx_v, deg_v):
    c = lax.axis_index("c")
    s = lax.axis_index("s")
    wid = s * NC + c

    def zero_body(t, carry):
        deg_v[0, pl.ds(t * 16, 16)] = jnp.zeros((16,), jnp.float32)
        return carry

    lax.fori_loop(0, N // 16, zero_body, 0)

    ones16 = jnp.ones((16,), jnp.float32)
    zeros16 = jnp.zeros((16,), jnp.int32)
    base = wid * EPW

    def stage_body(q, carry):
        pltpu.sync_copy(dst_hbm.at[pl.ds(base + q * 2000, 2000)], idx_v)

        def hist_body(t, c2):
            idx16 = idx_v[pl.ds(t * 16, 16)]
            plsc.addupdate_scatter(deg_v, [zeros16, idx16], ones16)
            return c2

        lax.fori_loop(0, 125, hist_body, 0)
        return carry

    lax.fori_loop(0, EPW // 2000, stage_body, 0)
    pltpu.sync_copy(deg_v, deg_out.at[wid])


def _deg_partials(dst):
    return pl.kernel(
        _deg_partials_body,
        out_type=jax.ShapeDtypeStruct((NW, 1, N), jnp.float32),
        mesh=_sc_mesh(),
        compiler_params=pltpu.CompilerParams(needs_layout_passes=False),
        scratch_types=[
            pltpu.VMEM((2000,), jnp.int32),
            pltpu.VMEM((1, N), jnp.float32),
        ],
    )(dst)


def _edge_pass_body(
    y_hbm, src_hbm, dst_hbm, out_hbm,
    sidx0, didx0, rows0, semi0, semg0,
    sidx1, didx1, rows1, semi1, semg1,
    sidxm, didxm, rowsm, semim, semgm,
    zrow, acc,
):
    c = lax.axis_index("c")
    s = lax.axis_index("s")
    wid = s * NC + c

    def zbuf_body(i, carry):
        for j in range(D // 16):
            zrow[i, pl.ds(j * 16, 16)] = jnp.zeros((16,), jnp.float32)
        return carry

    lax.fori_loop(0, ZR, zbuf_body, 0)

    @pl.when(s < NS - 1)
    def _():
        def zc(q, carry):
            pltpu.sync_copy(zrow, acc.at[pl.ds(s * 640 + q * ZR, ZR)])
            return carry

        lax.fori_loop(0, 640 // ZR, zc, 0)

    @pl.when(s == NS - 1)
    def _():
        def zc(q, carry):
            pltpu.sync_copy(zrow, acc.at[pl.ds((NS - 1) * 640 + q * ZR, ZR)])
            return carry

        lax.fori_loop(0, (N - (NS - 1) * 640) // ZR, zc, 0)

    plsc.subcore_barrier()

    base = wid * EPW
    bufs = (
        (sidx0, didx0, rows0, semi0, semg0),
        (sidx1, didx1, rows1, semi1, semg1),
    )

    def idx_start(t, buf):
        si, di, _, smi, _ = buf
        pltpu.async_copy(src_hbm.at[pl.ds(base + t * K, K)], si, smi)
        pltpu.async_copy(dst_hbm.at[pl.ds(base + t * K, K)], di, smi)

    def idx_wait(t, buf):
        si, di, _, smi, _ = buf
        pltpu.make_async_copy(src_hbm.at[pl.ds(base + t * K, K)], si, smi).wait()
        pltpu.make_async_copy(dst_hbm.at[pl.ds(base + t * K, K)], di, smi).wait()

    def gather_start(buf):
        si, _, ri, _, smg = buf
        pltpu.async_copy(y_hbm.at[si], ri, smg)

    def gather_wait(buf):
        si, _, ri, _, smg = buf
        pltpu.make_async_copy(y_hbm.at[si], ri, smg).wait()

    def scatter(buf):
        _, di, ri, _, _ = buf
        pltpu.sync_copy(ri, acc.at[di], add=True)

    mbase = base + NFULL * K

    def mini_idx_start():
        pltpu.async_copy(src_hbm.at[pl.ds(mbase, KM)], sidxm, semim)
        pltpu.async_copy(dst_hbm.at[pl.ds(mbase, KM)], didxm, semim)

    def mini_idx_wait():
        pltpu.make_async_copy(src_hbm.at[pl.ds(mbase, KM)], sidxm, semim).wait()
        pltpu.make_async_copy(dst_hbm.at[pl.ds(mbase, KM)], didxm, semim).wait()

    idx_start(0, bufs[0])
    idx_start(1, bufs[1])
    idx_wait(0, bufs[0])
    gather_start(bufs[0])

    def chunk_body(g, carry):
        for b in range(2):
            t = 2 * g + b
            cur = bufs[b]
            nxt = bufs[1 - b]
            idx_wait(t + 1, nxt)
            gather_start(nxt)
            gather_wait(cur)
            scatter(cur)

            @pl.when(t + 2 < NFULL)
            def _():
                idx_start(t + 2, cur)

        return carry

    lax.fori_loop(0, (NFULL - 2) // 2, chunk_body, 0)
    t0 = NFULL - 2
    mini_idx_start()
    idx_wait(t0 + 1, bufs[(t0 + 1) % 2])
    gather_start(bufs[(t0 + 1) % 2])
    gather_wait(bufs[t0 % 2])
    scatter(bufs[t0 % 2])
    mini_idx_wait()
    pltpu.async_copy(y_hbm.at[sidxm], rowsm, semgm)
    gather_wait(bufs[(t0 + 1) % 2])
    scatter(bufs[(t0 + 1) % 2])
    pltpu.make_async_copy(y_hbm.at[sidxm], rowsm, semgm).wait()
    pltpu.sync_copy(rowsm, acc.at[didxm], add=True)
    plsc.subcore_barrier()

    @pl.when(s < NS - 1)
    def _():
        pltpu.sync_copy(
            acc.at[pl.ds(s * 640, 640)], out_hbm.at[c, pl.ds(s * 640, 640)]
        )

    @pl.when(s == NS - 1)
    def _():
        pltpu.sync_copy(
            acc.at[pl.ds((NS - 1) * 640, N - (NS - 1) * 640)],
            out_hbm.at[c, pl.ds((NS - 1) * 640, N - (NS - 1) * 640)],
        )


def _edge_pass(y, src, dst):
    return pl.kernel(
        _edge_pass_body,
        out_type=jax.ShapeDtypeStruct((NC, N, D), jnp.float32),
        mesh=_sc_mesh(),
        compiler_params=pltpu.CompilerParams(needs_layout_passes=False),
        scratch_types=[
            pltpu.VMEM((K,), jnp.int32),
            pltpu.VMEM((K,), jnp.int32),
            pltpu.VMEM((K, D), jnp.float32),
            pltpu.SemaphoreType.DMA,
            pltpu.SemaphoreType.DMA,
            pltpu.VMEM((K,), jnp.int32),
            pltpu.VMEM((K,), jnp.int32),
            pltpu.VMEM((K, D), jnp.float32),
            pltpu.SemaphoreType.DMA,
            pltpu.SemaphoreType.DMA,
            pltpu.VMEM((KM,), jnp.int32),
            pltpu.VMEM((KM,), jnp.int32),
            pltpu.VMEM((KM, D), jnp.float32),
            pltpu.SemaphoreType.DMA,
            pltpu.SemaphoreType.DMA,
            pltpu.VMEM((ZR, D), jnp.float32),
            pltpu.VMEM_SHARED((N, D), jnp.float32),
        ],
    )(y, src, dst)


def _tc_norm_mm_body(x_ref, w_ref, degp_ref, y_ref, dinv_ref):
    deg = 1.0 + jnp.sum(degp_ref[...], axis=1, keepdims=True)
    dinv = lax.rsqrt(jnp.maximum(deg, 1e-12))
    xw = jnp.dot(x_ref[...], w_ref[...], preferred_element_type=jnp.float32)
    y_ref[...] = xw * dinv
    dinv_ref[...] = dinv


def _tc_norm_mm(x, w1, degp_t):
    return pl.pallas_call(
        _tc_norm_mm_body,
        grid=(GRID,),
        in_specs=[
            pl.BlockSpec((BLK, D), lambda i: (i, 0)),
            pl.BlockSpec((D, D), lambda i: (0, 0)),
            pl.BlockSpec((BLK, NW), lambda i: (i, 0)),
        ],
        out_specs=[
            pl.BlockSpec((BLK, D), lambda i: (i, 0)),
            pl.BlockSpec((BLK, 1), lambda i: (i, 0)),
        ],
        out_shape=[
            jax.ShapeDtypeStruct((N, D), jnp.float32),
            jax.ShapeDtypeStruct((N, 1), jnp.float32),
        ],
    )(x, w1, degp_t)


def _tc_mid_body(acc_ref, y1_ref, dinv_ref, b1_ref, w2_ref, y2_ref):
    a = acc_ref[0] + acc_ref[1] + y1_ref[...]
    dinv = dinv_ref[...]
    h = jnp.maximum(a * dinv + b1_ref[...], 0.0)
    y2_ref[...] = jnp.dot(h, w2_ref[...], preferred_element_type=jnp.float32) * dinv


def _tc_mid(acc, y1, dinv, b1_2d, w2):
    return pl.pallas_call(
        _tc_mid_body,
        grid=(GRID,),
        in_specs=[
            pl.BlockSpec((NC, BLK, D), lambda i: (0, i, 0)),
            pl.BlockSpec((BLK, D), lambda i: (i, 0)),
            pl.BlockSpec((BLK, 1), lambda i: (i, 0)),
            pl.BlockSpec((1, D), lambda i: (0, 0)),
            pl.BlockSpec((D, D), lambda i: (0, 0)),
        ],
        out_specs=pl.BlockSpec((BLK, D), lambda i: (i, 0)),
        out_shape=jax.ShapeDtypeStruct((N, D), jnp.float32),
    )(acc, y1, dinv, b1_2d, w2)


def _tc_pool_body(acc_ref, y2_ref, dinv_ref, b2_ref, batch_ref, out_ref, cnt_ref):
    i = pl.program_id(0)
    a = acc_ref[0] + acc_ref[1] + y2_ref[...]
    h = a * dinv_ref[...] + b2_ref[...]
    bt = batch_ref[0]
    oh_t = jnp.where(
        lax.broadcasted_iota(jnp.int32, (G, BLK), 0) == bt, 1.0, 0.0
    )
    sums = jnp.dot(oh_t, h, preferred_element_type=jnp.float32)
    cnts = jnp.dot(oh_t, jnp.ones((BLK, D), jnp.float32),
                   preferred_element_type=jnp.float32)

    @pl.when(i == 0)
    def _():
        out_ref[...] = jnp.zeros((G, D), jnp.float32)
        cnt_ref[...] = jnp.zeros((G, D), jnp.float32)

    out_ref[...] += sums
    cnt_ref[...] += cnts

    @pl.when(i == GRID - 1)
    def _():
        out_ref[...] = out_ref[...] / jnp.maximum(cnt_ref[...], 1.0)


def _tc_pool(acc, y2, dinv, b2_2d, batch_2d):
    return pl.pallas_call(
        _tc_pool_body,
        grid=(GRID,),
        in_specs=[
            pl.BlockSpec((NC, BLK, D), lambda i: (0, i, 0)),
            pl.BlockSpec((BLK, D), lambda i: (i, 0)),
            pl.BlockSpec((BLK, 1), lambda i: (i, 0)),
            pl.BlockSpec((1, D), lambda i: (0, 0)),
            pl.BlockSpec((1, 1, BLK), lambda i: (i, 0, 0)),
        ],
        out_specs=pl.BlockSpec((G, D), lambda i: (0, 0)),
        out_shape=jax.ShapeDtypeStruct((G, D), jnp.float32),
        scratch_shapes=[pltpu.VMEM((G, D), jnp.float32)],
    )(acc, y2, dinv, b2_2d, batch_2d)


def kernel(x, edge_index, batch, W1, b1, W2, b2):
    src = edge_index[0]
    dst = edge_index[1]

    degp = _deg_partials(dst)
    degp_t = degp.reshape(NW, N).T

    y1, dinv = _tc_norm_mm(x, W1, degp_t)
    acc1 = _edge_pass(y1, src, dst)
    y2 = _tc_mid(acc1, y1, dinv, b1.reshape(1, D), W2)
    acc2 = _edge_pass(y2, src, dst)
    return _tc_pool(acc2, y2, dinv, b2.reshape(1, D), batch.reshape(GRID, 1, BLK))
